# baseline (speedup 1.0000x reference)
import jax
import jax.numpy as jnp
from jax.experimental import pallas as pl

_B, _C, _D = 16384, 26, 128
_R = 512
_G = _B // _R


def _body(mask_ref, x_ref, o_ref):
    m = mask_ref[0, 0, :]
    o_ref[...] = jnp.where(m[:, None, None] != 0, x_ref[...], 0.0)


def kernel(x, mask):
    m3 = mask.astype(jnp.int32).reshape(_G, 1, _R)
    return pl.pallas_call(
        _body,
        grid=(_G,),
        in_specs=[
            pl.BlockSpec((1, 1, _R), lambda i: (i, 0, 0)),
            pl.BlockSpec((_R, _C, _D), lambda i: (i, 0, 0)),
        ],
        out_specs=pl.BlockSpec((_R, _C, _D), lambda i: (i, 0, 0)),
        out_shape=jax.ShapeDtypeStruct((_B, _C, _D), jnp.float32),
    )(m3, x)


# speedup vs baseline: 1.0385x; 1.0385x over previous
import functools

import jax
import jax.numpy as jnp
from jax import lax
from jax.experimental import pallas as pl
from jax.experimental.pallas import tpu as pltpu
from jax.experimental.pallas import tpu_sc as plsc

_B, _C, _D = 16384, 26, 128
_NC, _NS = 2, 16
_NW = _NC * _NS
_RPW = _B // _NW
_NG = _RPW // 16
_K = 8
_Z = 8
_CELL = _RPW + 16
_TRASH = _RPW + 24
_IDXN = _RPW + 40


def _sc_body(x_hbm, mask_hbm, zrows_hbm, out_hbm,
             mask_v, idxt, idxf, bufa, bufb, zbuf,
             gsema, gsemb, ssema, ssemb, zsema, zsemb, msem):
    wid = lax.axis_index("s") * _NC + lax.axis_index("c")
    base = wid * _RPW

    pltpu.make_async_copy(mask_hbm.at[pl.ds(base, _RPW)], mask_v, msem).start()
    pltpu.make_async_copy(zrows_hbm, zbuf, zsema).start()
    pltpu.make_async_copy(mask_hbm.at[pl.ds(base, _RPW)], mask_v, msem).wait()
    pltpu.make_async_copy(zrows_hbm, zbuf, zsema).wait()

    lanes = lax.iota(jnp.int32, 16)
    dnums = lax.GatherDimensionNumbers(
        offset_dims=(), collapsed_slice_dims=(0,), start_index_map=(0,))

    def _lane_gather(v, idx):
        return lax.gather(v, idx[:, None], dnums, slice_sizes=(1,),
                          mode=lax.GatherScatterMode.PROMISE_IN_BOUNDS)

    def _incl_cumsum(v):
        s = v
        for sh in (1, 2, 4, 8):
            g = _lane_gather(s, jnp.maximum(lanes - sh, 0))
            s = jnp.where(lanes >= sh, s + g, s)
        return s

    n_t = jnp.int32(0)
    n_f = jnp.int32(0)
    trash = lanes + _TRASH
    for g in range(_NG):
        mv = mask_v[pl.ds(g * 16, 16)]
        rows = lanes + (base + g * 16)
        m_i = jnp.where(mv != 0, 1, 0)
        incl = _incl_cumsum(m_i)
        pos_t = jnp.where(mv != 0, n_t + incl - 1, trash)
        pos_f = jnp.where(mv != 0, trash, n_f + lanes - incl)
        plsc.store_scatter(idxt, [pos_t], rows)
        plsc.store_scatter(idxf, [pos_f], rows)
        plsc.store_scatter(idxt, [jnp.where(mv != 0, _CELL, trash)], rows)
        plsc.store_scatter(idxf, [jnp.where(mv != 0, trash, _CELL)], rows)
        cnt = incl[15]
        n_t = n_t + cnt
        n_f = n_f + (16 - cnt)
    cell_t = idxt[pl.ds(_CELL, 16)]
    cell_f = idxf[pl.ds(_CELL, 16)]
    idxt[pl.ds(n_t, 16)] = jnp.full((16,), cell_t[0], jnp.int32)
    idxf[pl.ds(n_f, 16)] = jnp.full((16,), cell_f[0], jnp.int32)

    ncht = (n_t + (_K - 1)) // _K
    nchf = (n_f + (_Z - 1)) // _Z
    pairs = jnp.maximum((ncht + 1) // 2, (nchf + 1) // 2)

    def pair_body(p, carry):
        c0 = 2 * p
        c1 = c0 + 1

        @pl.when(c0 < ncht)
        def _():
            pltpu.make_async_copy(
                x_hbm.at[idxt.at[pl.ds(c0 * _K, _K)]], bufa, gsema).start()

        @pl.when(c1 < ncht)
        def _():
            pltpu.make_async_copy(
                x_hbm.at[idxt.at[pl.ds(c1 * _K, _K)]], bufb, gsemb).start()

        @pl.when(c0 < nchf)
        def _():
            pltpu.make_async_copy(
                zbuf, out_hbm.at[idxf.at[pl.ds(c0 * _Z, _Z)]], zsema).start()

        @pl.when(c1 < nchf)
        def _():
            pltpu.make_async_copy(
                zbuf, out_hbm.at[idxf.at[pl.ds(c1 * _Z, _Z)]], zsemb).start()

        @pl.when(c0 < ncht)
        def _():
            pltpu.make_async_copy(
                x_hbm.at[idxt.at[pl.ds(c0 * _K, _K)]], bufa, gsema).wait()
            pltpu.make_async_copy(
                bufa, out_hbm.at[idxt.at[pl.ds(c0 * _K, _K)]], ssema).start()

        @pl.when(c1 < ncht)
        def _():
            pltpu.make_async_copy(
                x_hbm.at[idxt.at[pl.ds(c1 * _K, _K)]], bufb, gsemb).wait()
            pltpu.make_async_copy(
                bufb, out_hbm.at[idxt.at[pl.ds(c1 * _K, _K)]], ssemb).start()

        @pl.when(c0 < ncht)
        def _():
            pltpu.make_async_copy(
                bufa, out_hbm.at[idxt.at[pl.ds(c0 * _K, _K)]], ssema).wait()

        @pl.when(c1 < ncht)
        def _():
            pltpu.make_async_copy(
                bufb, out_hbm.at[idxt.at[pl.ds(c1 * _K, _K)]], ssemb).wait()

        @pl.when(c0 < nchf)
        def _():
            pltpu.make_async_copy(
                zbuf, out_hbm.at[idxf.at[pl.ds(c0 * _Z, _Z)]], zsema).wait()

        @pl.when(c1 < nchf)
        def _():
            pltpu.make_async_copy(
                zbuf, out_hbm.at[idxf.at[pl.ds(c1 * _Z, _Z)]], zsemb).wait()

        return carry

    lax.fori_loop(0, pairs, pair_body, jnp.int32(0))


_sc_call = functools.partial(
    pl.kernel,
    out_type=jax.ShapeDtypeStruct((_B, _C, _D), jnp.float32),
    mesh=plsc.VectorSubcoreMesh(core_axis_name="c", subcore_axis_name="s"),
    compiler_params=pltpu.CompilerParams(needs_layout_passes=False),
    scratch_types=[
        pltpu.VMEM((_RPW,), jnp.int32),
        pltpu.VMEM((_IDXN,), jnp.int32),
        pltpu.VMEM((_IDXN,), jnp.int32),
        pltpu.VMEM((_K, _C, _D), jnp.float32),
        pltpu.VMEM((_K, _C, _D), jnp.float32),
        pltpu.VMEM((_Z, _C, _D), jnp.float32),
        pltpu.SemaphoreType.DMA,
        pltpu.SemaphoreType.DMA,
        pltpu.SemaphoreType.DMA,
        pltpu.SemaphoreType.DMA,
        pltpu.SemaphoreType.DMA,
        pltpu.SemaphoreType.DMA,
        pltpu.SemaphoreType.DMA,
    ],
)(_sc_body)


def kernel(x, mask):
    mask_i32 = mask.astype(jnp.int32)
    zrows = jnp.zeros((_Z, _C, _D), jnp.float32)
    return _sc_call(x, mask_i32, zrows)


# speedup vs baseline: 2.7594x; 2.6572x over previous
import functools

import jax
import jax.numpy as jnp
from jax import lax
from jax.experimental import pallas as pl
from jax.experimental.pallas import tpu as pltpu
from jax.experimental.pallas import tpu_sc as plsc

_B, _C, _D = 16384, 26, 128
_NC, _NS = 2, 16
_NW = _NC * _NS
_RPW = _B // _NW
_NG = _RPW // 16
_K = 128
_CELL = _RPW + 16
_TRASH = _RPW + 32
_IDXN = _RPW + 48
_FULLN = _C * (_RPW + 16) + _K + 16


def _sc_body(x_hbm, mask_hbm, zrows_hbm, out_hbm,
             mask_v, idxt, idxf, fullt, fullf, bufa, bufb, zbuf,
             gsema, gsemb, ssema, ssemb, zsema, zsemb, msem):
    wid = lax.axis_index("s") * _NC + lax.axis_index("c")
    base = wid * _RPW

    pltpu.make_async_copy(mask_hbm.at[pl.ds(base, _RPW)], mask_v, msem).start()
    pltpu.make_async_copy(zrows_hbm, zbuf, zsema).start()
    pltpu.make_async_copy(mask_hbm.at[pl.ds(base, _RPW)], mask_v, msem).wait()
    pltpu.make_async_copy(zrows_hbm, zbuf, zsema).wait()

    lanes = lax.iota(jnp.int32, 16)
    dnums = lax.GatherDimensionNumbers(
        offset_dims=(), collapsed_slice_dims=(0,), start_index_map=(0,))

    def _lane_gather(v, idx):
        return lax.gather(v, idx[:, None], dnums, slice_sizes=(1,),
                          mode=lax.GatherScatterMode.PROMISE_IN_BOUNDS)

    def _incl_cumsum(v):
        s = v
        for sh in (1, 2, 4, 8):
            g = _lane_gather(s, jnp.maximum(lanes - sh, 0))
            s = jnp.where(lanes >= sh, s + g, s)
        return s

    n_t = jnp.int32(0)
    n_f = jnp.int32(0)
    trash = lanes + _TRASH
    for g in range(_NG):
        mv = mask_v[pl.ds(g * 16, 16)]
        rows = lanes + (base + g * 16)
        m_i = jnp.where(mv != 0, 1, 0)
        incl = _incl_cumsum(m_i)
        pos_t = jnp.where(mv != 0, n_t + incl - 1, trash)
        pos_f = jnp.where(mv != 0, trash, n_f + lanes - incl)
        plsc.store_scatter(idxt, [pos_t], rows)
        plsc.store_scatter(idxf, [pos_f], rows)
        plsc.store_scatter(idxt, [jnp.where(mv != 0, _CELL, trash)], rows)
        plsc.store_scatter(idxf, [jnp.where(mv != 0, trash, _CELL)], rows)
        cnt = incl[15]
        n_t = n_t + cnt
        n_f = n_f + (16 - cnt)
    cell_t = jnp.full((16,), idxt[pl.ds(_CELL, 16)][0], jnp.int32)
    cell_f = jnp.full((16,), idxf[pl.ds(_CELL, 16)][0], jnp.int32)
    idxt[pl.ds(n_t, 16)] = cell_t
    idxf[pl.ds(n_f, 16)] = cell_f
    ntp = ((n_t + 15) >> 4) << 4
    nfp = ((n_f + 15) >> 4) << 4

    def _replicate(src, dst, npad):
        ngroups = npad >> 4

        def body(i, carry):
            c = i // ngroups
            g = i - c * ngroups
            v = src[pl.ds(g * 16, 16)]
            dst[pl.ds(c * npad + g * 16, 16)] = v + c * _B
            return carry

        lax.fori_loop(0, _C * ngroups, body, jnp.int32(0))

    _replicate(idxt, fullt, ntp)
    _replicate(idxf, fullf, nfp)
    nft = _C * ntp
    nff = _C * nfp
    for j in range(_K // 16):
        fullt[pl.ds(nft + j * 16, 16)] = cell_t
        fullf[pl.ds(nff + j * 16, 16)] = cell_f

    ncht = (nft + (_K - 1)) // _K
    nchf = (nff + (_K - 1)) // _K
    pairs = jnp.maximum((ncht + 1) // 2, (nchf + 1) // 2)

    def pair_body(p, carry):
        c0 = 2 * p
        c1 = c0 + 1

        @pl.when(c0 < ncht)
        def _():
            pltpu.make_async_copy(
                x_hbm.at[fullt.at[pl.ds(c0 * _K, _K)]], bufa, gsema).start()

        @pl.when(c1 < ncht)
        def _():
            pltpu.make_async_copy(
                x_hbm.at[fullt.at[pl.ds(c1 * _K, _K)]], bufb, gsemb).start()

        @pl.when(c0 < nchf)
        def _():
            pltpu.make_async_copy(
                zbuf, out_hbm.at[fullf.at[pl.ds(c0 * _K, _K)]], zsema).start()

        @pl.when(c1 < nchf)
        def _():
            pltpu.make_async_copy(
                zbuf, out_hbm.at[fullf.at[pl.ds(c1 * _K, _K)]], zsemb).start()

        @pl.when(c0 < ncht)
        def _():
            pltpu.make_async_copy(
                x_hbm.at[fullt.at[pl.ds(c0 * _K, _K)]], bufa, gsema).wait()
            pltpu.make_async_copy(
                bufa, out_hbm.at[fullt.at[pl.ds(c0 * _K, _K)]], ssema).start()

        @pl.when(c1 < ncht)
        def _():
            pltpu.make_async_copy(
                x_hbm.at[fullt.at[pl.ds(c1 * _K, _K)]], bufb, gsemb).wait()
            pltpu.make_async_copy(
                bufb, out_hbm.at[fullt.at[pl.ds(c1 * _K, _K)]], ssemb).start()

        @pl.when(c0 < ncht)
        def _():
            pltpu.make_async_copy(
                bufa, out_hbm.at[fullt.at[pl.ds(c0 * _K, _K)]], ssema).wait()

        @pl.when(c1 < ncht)
        def _():
            pltpu.make_async_copy(
                bufb, out_hbm.at[fullt.at[pl.ds(c1 * _K, _K)]], ssemb).wait()

        @pl.when(c0 < nchf)
        def _():
            pltpu.make_async_copy(
                zbuf, out_hbm.at[fullf.at[pl.ds(c0 * _K, _K)]], zsema).wait()

        @pl.when(c1 < nchf)
        def _():
            pltpu.make_async_copy(
                zbuf, out_hbm.at[fullf.at[pl.ds(c1 * _K, _K)]], zsemb).wait()

        return carry

    lax.fori_loop(0, pairs, pair_body, jnp.int32(0))


_sc_call = functools.partial(
    pl.kernel,
    out_type=jax.ShapeDtypeStruct((_C * _B, _D), jnp.float32),
    mesh=plsc.VectorSubcoreMesh(core_axis_name="c", subcore_axis_name="s"),
    compiler_params=pltpu.CompilerParams(needs_layout_passes=False),
    scratch_types=[
        pltpu.VMEM((_RPW,), jnp.int32),
        pltpu.VMEM((_IDXN,), jnp.int32),
        pltpu.VMEM((_IDXN,), jnp.int32),
        pltpu.VMEM((_FULLN,), jnp.int32),
        pltpu.VMEM((_FULLN,), jnp.int32),
        pltpu.VMEM((_K, _D), jnp.float32),
        pltpu.VMEM((_K, _D), jnp.float32),
        pltpu.VMEM((_K, _D), jnp.float32),
        pltpu.SemaphoreType.DMA,
        pltpu.SemaphoreType.DMA,
        pltpu.SemaphoreType.DMA,
        pltpu.SemaphoreType.DMA,
        pltpu.SemaphoreType.DMA,
        pltpu.SemaphoreType.DMA,
        pltpu.SemaphoreType.DMA,
    ],
)(_sc_body)


def kernel(x, mask):
    x2 = jnp.transpose(x, (1, 0, 2)).reshape(_C * _B, _D)
    mask_i32 = mask.astype(jnp.int32)
    zrows = jnp.zeros((_K, _D), jnp.float32)
    y2 = _sc_call(x2, mask_i32, zrows)
    return jnp.transpose(y2.reshape(_C, _B, _D), (1, 0, 2))
